# baseline (speedup 1.0000x reference)
import functools

import jax
import jax.numpy as jnp
from jax import lax
from jax.experimental import pallas as pl
from jax.experimental.pallas import tpu as pltpu
from jax.experimental.pallas import tpu_sc as plsc

_N = 10000
_E = 320000
_D_IN = 128
_HID = 64
_OUT = 16

_NC = 2
_NS = 16
_NW = _NC * _NS
_C = 128
_K = 4
_EP = 327680
_NJUNK = 1024
_NPAD = _N + _NJUNK
_ROWS = _EP // _C

_DEGF = 8


def _make_spmm(F, mode):
  gather = mode != "deg"
  dual = mode == "dual"
  cpw = _ROWS // (_NS if dual else _NW)
  nbatch = cpw // _K
  npair = nbatch // 2
  rpt = _NPAD // _NS
  mesh = plsc.VectorSubcoreMesh(core_axis_name="c", subcore_axis_name="s")
  scratch = [
      pltpu.VMEM((2, 2 * _K, _C), jnp.int32),
      pltpu.VMEM((2, 2 * _K, _C), jnp.int32),
      pltpu.VMEM((2, _K, _C, F), jnp.float32),
      pltpu.VMEM_SHARED((_NPAD, F), jnp.float32),
      pltpu.SemaphoreType.DMA,
      pltpu.SemaphoreType.DMA,
      pltpu.SemaphoreType.DMA,
      pltpu.SemaphoreType.DMA,
      pltpu.SemaphoreType.DMA,
  ]

  @functools.partial(
      pl.kernel,
      out_type=jax.ShapeDtypeStruct((_NC, _NPAD, F), jnp.float32),
      mesh=mesh,
      scratch_types=scratch,
      compiler_params=pltpu.CompilerParams(use_tc_tiling_on_sc=False),
  )
  def spmm(table_hbm, src_hbm, dst_hbm, zeros_hbm, out_hbm,
           src_b, dst_b, rows_b, acc_sh, g0, g1, s0, s1, isem):
    c = lax.axis_index("c")
    s = lax.axis_index("s")
    gsem = (g0, g1)
    ssem = (s0, s1)
    row0 = (s if dual else c * _NS + s) * cpw

    pltpu.sync_copy(zeros_hbm.at[pl.ds(s * rpt, rpt)],
                    acc_sh.at[pl.ds(s * rpt, rpt)])

    if not gather:
      pltpu.sync_copy(table_hbm, rows_b.at[0, 0])

    def idx_pair(pair, par):
      r = row0 + pair * 2 * _K
      dsts = pltpu.make_async_copy(dst_hbm.at[pl.ds(r, 2 * _K)],
                                   dst_b.at[par], isem)
      if not gather:
        return (dsts,)
      if dual:
        srcs = pltpu.make_async_copy(src_hbm.at[c, pl.ds(r, 2 * _K)],
                                     src_b.at[par], isem)
      else:
        srcs = pltpu.make_async_copy(src_hbm.at[pl.ds(r, 2 * _K)],
                                     src_b.at[par], isem)
      return (dsts, srcs)

    def idx_set(st, batch, par):
      r = row0 + batch * _K
      dsts = pltpu.make_async_copy(dst_hbm.at[pl.ds(r, _K)],
                                   dst_b.at[par, pl.ds(st * _K, _K)], isem)
      if dual:
        srcs = pltpu.make_async_copy(src_hbm.at[c, pl.ds(r, _K)],
                                     src_b.at[par, pl.ds(st * _K, _K)], isem)
      else:
        srcs = pltpu.make_async_copy(src_hbm.at[pl.ds(r, _K)],
                                     src_b.at[par, pl.ds(st * _K, _K)], isem)
      return (dsts, srcs)

    def fire_gathers(st, par):
      if gather:
        for b in range(_K):
          pltpu.async_copy(table_hbm.at[src_b.at[par, st * _K + b]],
                           rows_b.at[st, b], gsem[st])

    def drain_gathers(st, par):
      if gather:
        for b in range(_K):
          pltpu.make_async_copy(table_hbm.at[src_b.at[par, st * _K + b]],
                                rows_b.at[st, b], gsem[st]).wait()

    def fire_scatters(st, par):
      for b in range(_K):
        rows = rows_b.at[st, b] if gather else rows_b.at[0, 0]
        pltpu.async_copy(rows, acc_sh.at[dst_b.at[par, st * _K + b]],
                         ssem[st], add=True)

    def drain_scatters(st, par):
      for b in range(_K):
        rows = rows_b.at[st, b] if gather else rows_b.at[0, 0]
        pltpu.make_async_copy(rows, acc_sh.at[dst_b.at[par, st * _K + b]],
                              ssem[st]).wait()

    plsc.subcore_barrier()

    if gather:
      def loadw(st, batch):
        descs = idx_set(st, batch, 0)
        for d in descs:
          d.start()
        for d in descs:
          d.wait()

      loadw(0, 0)
      fire_gathers(0, 0)
      loadw(1, 1)
      fire_gathers(1, 0)

      def body(j, carry):
        drain_gathers(0, 0)
        fire_scatters(0, 0)
        drain_gathers(1, 0)
        fire_scatters(1, 0)

        @pl.when(j < npair - 1)
        def _next():
          drain_scatters(0, 0)
          loadw(0, 2 * j + 2)
          fire_gathers(0, 0)
          drain_scatters(1, 0)
          loadw(1, 2 * j + 3)
          fire_gathers(1, 0)

        return carry

      lax.fori_loop(0, npair, body, 0)
      drain_scatters(0, 0)
      drain_scatters(1, 0)
    else:
      descs0 = idx_pair(0, 0)
      for d in descs0:
        d.start()
      for d in descs0:
        d.wait()

      def body(j, carry):
        p = j % 2
        q = 1 - p
        fire_scatters(0, p)
        fire_scatters(1, p)

        @pl.when(j < npair - 1)
        def _next():
          descs = idx_pair(j + 1, q)
          for d in descs:
            d.start()
          drain_scatters(0, p)
          drain_scatters(1, p)
          for d in descs:
            d.wait()

        return carry

      lax.fori_loop(0, npair, body, 0)
      pf = (npair - 1) % 2
      drain_scatters(0, pf)
      drain_scatters(1, pf)

    plsc.subcore_barrier()
    pltpu.sync_copy(acc_sh.at[pl.ds(s * rpt, rpt)],
                    out_hbm.at[c, pl.ds(s * rpt, rpt)])

  return spmm


_spmm_dual = _make_spmm(_HID, "dual")
_spmm64 = _make_spmm(_HID, "split")
_deg_hist = _make_spmm(_DEGF, "deg")


def _norm(z):
  n = jnp.sqrt(jnp.sum(z * z, axis=1, keepdims=True))
  return z / jnp.maximum(n, 1e-12)


def _leaky(z):
  return jnp.where(z >= 0, z, 0.01 * z)


def _softmax_head(alpha_row, gamma_s):
  m = jnp.max(alpha_row)
  e = jnp.exp(alpha_row - m)
  return gamma_s * e / jnp.sum(e)


def _tc1_body(x_ref, w1_ref, b1_ref, degp_ref, y_ref, ds_ref):
  deg = degp_ref[0, 0:_N, 0:1] + degp_ref[1, 0:_N, 0:1]
  ds = jax.lax.rsqrt(jnp.maximum(deg, 1.0))
  h = jnp.dot(x_ref[...], w1_ref[...],
              preferred_element_type=jnp.float32) + b1_ref[...]
  y_ref[...] = ds * h
  ds_ref[...] = ds


_RB = 2000


def _tc2_body(pa_ref, pb_ref, ds_ref, w0_ref, b0_ref, w1_ref, b1_ref,
              w2_ref, b2_ref, al_ref, g_ref, yc_ref, dp_ref, acc_ref):
  a = _softmax_head(al_ref[0, :], g_ref[0, 0])
  ds = ds_ref[...]
  z0 = _leaky(_norm(ds * pa_ref[0]))
  head0 = jnp.dot(z0, w0_ref[...], preferred_element_type=jnp.float32) + b0_ref[...]
  sb = pb_ref[0] + pb_ref[1]
  z1 = _leaky(_norm(0.5 * ds * sb))
  head1 = jnp.dot(z1, w1_ref[...], preferred_element_type=jnp.float32) + b1_ref[...]
  x_cat = jnp.concatenate([z0, z1], axis=1)
  dp = jnp.dot(x_cat, w2_ref[...], preferred_element_type=jnp.float32) + b2_ref[...]
  dp_ref[...] = dp
  yc_ref[...] = ds * dp
  acc_ref[...] = a[0] * _norm(head0) + a[1] * _norm(head1)


def _tc4_body(pc_ref, ds_ref, dp_ref, w_ref, b_ref, wd_ref, bd_ref,
              al_ref, g_ref, acc_in_ref, y_ref):
  a = _softmax_head(al_ref[0, :], g_ref[0, 0])
  sc = pc_ref[0, 0:_N, :] + pc_ref[1, 0:_N, :]
  ds = ds_ref[...]
  z = ds * sc + dp_ref[...]
  z2 = _leaky(_norm(z))
  head = jnp.dot(z2, w_ref[...], preferred_element_type=jnp.float32) + b_ref[...]
  out = acc_in_ref[...] + a[2] * _norm(head)
  out = out + a[3] * (ds * wd_ref[...] + bd_ref[...])
  m = jnp.max(out, axis=1, keepdims=True)
  sh = out - m
  y_ref[...] = sh - jnp.log(jnp.sum(jnp.exp(sh), axis=1, keepdims=True))


def _tc(body, out_shapes):
  return pl.pallas_call(body, out_shape=out_shapes)


def kernel(x, edge_index, W1, b1, W2, b2, W_out, b_out, Wd, bd, alpha, gamma):
  f32 = jnp.float32
  i32 = jnp.int32
  pad = _EP - _E
  src = jnp.concatenate([edge_index[0], jnp.arange(pad, dtype=i32) % _N])
  dst = jnp.concatenate(
      [edge_index[1], _N + (jnp.arange(pad, dtype=i32) % _NJUNK)])
  src = src.reshape(_ROWS, _C)
  dst = dst.reshape(_ROWS, _C)
  src_dual = jnp.stack([2 * src, 2 * src + 1])
  zeros_deg = jnp.zeros((_NPAD, _DEGF), f32)
  zeros64 = jnp.zeros((_NPAD, _HID), f32)
  ones_rows = jnp.ones((_C, _DEGF), f32)
  al_row = alpha.reshape(1, 4).astype(f32)
  g_sc = jnp.reshape(gamma, (1, 1)).astype(f32)

  degp = _deg_hist(ones_rows, src, dst, zeros_deg)

  y128, ds = _tc(_tc1_body, (
      jax.ShapeDtypeStruct((_N, _D_IN), f32),
      jax.ShapeDtypeStruct((_N, 1), f32),
  ))(x, W1, b1.reshape(1, _D_IN), degp)

  pa = _spmm_dual(y128.reshape(2 * _N, _HID), src_dual, dst, zeros64)

  pb = _spmm64(pa[1, 0:_N], src, dst, zeros64)

  nb = _N // _RB
  full = lambda *shape: pl.BlockSpec(shape, lambda i: (0,) * len(shape))
  yc, dp, acc12 = pl.pallas_call(
      _tc2_body,
      grid=(nb,),
      in_specs=[
          pl.BlockSpec((2, _RB, _HID), lambda i: (0, i, 0)),
          pl.BlockSpec((2, _RB, _HID), lambda i: (0, i, 0)),
          pl.BlockSpec((_RB, 1), lambda i: (i, 0)),
          full(_HID, _OUT), full(1, _OUT),
          full(_HID, _OUT), full(1, _OUT),
          full(2 * _HID, _HID), full(1, _HID),
          full(1, 4), full(1, 1),
      ],
      out_specs=[
          pl.BlockSpec((_RB, _HID), lambda i: (i, 0)),
          pl.BlockSpec((_RB, _HID), lambda i: (i, 0)),
          pl.BlockSpec((_RB, _OUT), lambda i: (i, 0)),
      ],
      out_shape=[
          jax.ShapeDtypeStruct((_N, _HID), f32),
          jax.ShapeDtypeStruct((_N, _HID), f32),
          jax.ShapeDtypeStruct((_N, _OUT), f32),
      ],
  )(pa, pb, ds, W_out[0], b_out[0].reshape(1, _OUT), W_out[1],
    b_out[1].reshape(1, _OUT), W2, b2.reshape(1, _HID), al_row, g_sc)

  pc = _spmm64(yc, src, dst, zeros64)

  y_hat = _tc(_tc4_body, jax.ShapeDtypeStruct((_N, _OUT), f32))(
      pc, ds, dp, W_out[0], b_out[0].reshape(1, _OUT), Wd,
      bd.reshape(1, _OUT), al_row, g_sc, acc12)

  return y_hat


# speedup vs baseline: 1.0984x; 1.0984x over previous
import functools

import jax
import jax.numpy as jnp
from jax import lax
from jax.experimental import pallas as pl
from jax.experimental.pallas import tpu as pltpu
from jax.experimental.pallas import tpu_sc as plsc

_N = 10000
_E = 320000
_D_IN = 128
_HID = 64
_OUT = 16

_NC = 2
_NS = 16
_NW = _NC * _NS
_C = 128
_K = 4
_EP = 327680
_NJUNK = 1024
_NPAD = _N + _NJUNK
_ROWS = _EP // _C

_DEGF = 8


def _make_spmm(F, mode):
  gather = mode != "deg"
  dual = mode == "dual"
  cpw = _ROWS // (_NS if dual else _NW)
  nbatch = cpw // _K
  npair = nbatch // 2
  rpt = _NPAD // _NS
  mesh = plsc.VectorSubcoreMesh(core_axis_name="c", subcore_axis_name="s")
  if gather:
    idx_shape = (2, _K, _C)
  else:
    idx_shape = (2, 2 * _K, _C)
  scratch = [
      pltpu.VMEM(idx_shape, jnp.int32),
      pltpu.VMEM(idx_shape, jnp.int32),
      pltpu.VMEM((2, _K, _C, F), jnp.float32),
      pltpu.VMEM_SHARED((_NPAD, F), jnp.float32),
      pltpu.SemaphoreType.DMA,
      pltpu.SemaphoreType.DMA,
      pltpu.SemaphoreType.DMA,
      pltpu.SemaphoreType.DMA,
      pltpu.SemaphoreType.DMA,
  ]

  @functools.partial(
      pl.kernel,
      out_type=jax.ShapeDtypeStruct((_NC, _NPAD, F), jnp.float32),
      mesh=mesh,
      scratch_types=scratch,
      compiler_params=pltpu.CompilerParams(use_tc_tiling_on_sc=False),
  )
  def spmm(table_hbm, src_hbm, dst_hbm, zeros_hbm, out_hbm,
           src_b, dst_b, rows_b, acc_sh, g0, g1, s0, s1, isem):
    c = lax.axis_index("c")
    s = lax.axis_index("s")
    gsem = (g0, g1)
    ssem = (s0, s1)
    row0 = (s if dual else c * _NS + s) * cpw

    pltpu.sync_copy(zeros_hbm.at[pl.ds(s * rpt, rpt)],
                    acc_sh.at[pl.ds(s * rpt, rpt)])

    if not gather:
      pltpu.sync_copy(table_hbm, rows_b.at[0, 0])

    def idx_pair(pair, par):
      r = row0 + pair * 2 * _K
      dsts = pltpu.make_async_copy(dst_hbm.at[pl.ds(r, 2 * _K)],
                                   dst_b.at[par], isem)
      if not gather:
        return (dsts,)
      if dual:
        srcs = pltpu.make_async_copy(src_hbm.at[c, pl.ds(r, 2 * _K)],
                                     src_b.at[par], isem)
      else:
        srcs = pltpu.make_async_copy(src_hbm.at[pl.ds(r, 2 * _K)],
                                     src_b.at[par], isem)
      return (dsts, srcs)

    def fire_gathers(st, par=None):
      if gather:
        for b in range(_K):
          pltpu.async_copy(table_hbm.at[src_b.at[st, b]],
                           rows_b.at[st, b], gsem[st])

    def drain_gathers(st, par=None):
      if gather:
        for b in range(_K):
          pltpu.make_async_copy(table_hbm.at[src_b.at[st, b]],
                                rows_b.at[st, b], gsem[st]).wait()

    def fire_scatters(st, par=None):
      for b in range(_K):
        if gather:
          pltpu.async_copy(rows_b.at[st, b], acc_sh.at[dst_b.at[st, b]],
                           ssem[st], add=True)
        else:
          pltpu.async_copy(rows_b.at[0, 0],
                           acc_sh.at[dst_b.at[par, st * _K + b]],
                           ssem[st], add=True)

    def drain_scatters(st, par=None):
      for b in range(_K):
        if gather:
          pltpu.make_async_copy(rows_b.at[st, b], acc_sh.at[dst_b.at[st, b]],
                                ssem[st]).wait()
        else:
          pltpu.make_async_copy(rows_b.at[0, 0],
                                acc_sh.at[dst_b.at[par, st * _K + b]],
                                ssem[st]).wait()

    plsc.subcore_barrier()

    if gather:
      def load_idx(st, batch):
        r = row0 + batch * _K
        pltpu.sync_copy(dst_hbm.at[pl.ds(r, _K)], dst_b.at[st])
        if dual:
          pltpu.sync_copy(src_hbm.at[c, pl.ds(r, _K)], src_b.at[st])
        else:
          pltpu.sync_copy(src_hbm.at[pl.ds(r, _K)], src_b.at[st])

      load_idx(0, 0)
      fire_gathers(0)
      load_idx(1, 1)
      fire_gathers(1)

      def body(j, carry):
        drain_gathers(0)
        fire_scatters(0)
        drain_gathers(1)
        fire_scatters(1)

        @pl.when(j < npair - 1)
        def _next():
          drain_scatters(0)
          load_idx(0, 2 * j + 2)
          fire_gathers(0)
          drain_scatters(1)
          load_idx(1, 2 * j + 3)
          fire_gathers(1)

        return carry

      lax.fori_loop(0, npair, body, 0)
      drain_scatters(0)
      drain_scatters(1)
    else:
      descs0 = idx_pair(0, 0)
      for d in descs0:
        d.start()
      for d in descs0:
        d.wait()

      def body(j, carry):
        p = j % 2
        q = 1 - p
        fire_scatters(0, p)
        fire_scatters(1, p)

        @pl.when(j < npair - 1)
        def _next():
          descs = idx_pair(j + 1, q)
          for d in descs:
            d.start()
          drain_scatters(0, p)
          drain_scatters(1, p)
          for d in descs:
            d.wait()

        return carry

      lax.fori_loop(0, npair, body, 0)
      pf = (npair - 1) % 2
      drain_scatters(0, pf)
      drain_scatters(1, pf)

    plsc.subcore_barrier()
    pltpu.sync_copy(acc_sh.at[pl.ds(s * rpt, rpt)],
                    out_hbm.at[c, pl.ds(s * rpt, rpt)])

  return spmm


_spmm_dual = _make_spmm(_HID, "dual")
_spmm64 = _make_spmm(_HID, "split")
_deg_hist = _make_spmm(_DEGF, "deg")


def _norm(z):
  n = jnp.sqrt(jnp.sum(z * z, axis=1, keepdims=True))
  return z / jnp.maximum(n, 1e-12)


def _leaky(z):
  return jnp.where(z >= 0, z, 0.01 * z)


def _softmax_head(alpha_row, gamma_s):
  m = jnp.max(alpha_row)
  e = jnp.exp(alpha_row - m)
  return gamma_s * e / jnp.sum(e)


def _tc1_body(x_ref, w1_ref, b1_ref, degp_ref, y_ref, ds_ref):
  deg = degp_ref[0, 0:_N, 0:1] + degp_ref[1, 0:_N, 0:1]
  ds = jax.lax.rsqrt(jnp.maximum(deg, 1.0))
  h = jnp.dot(x_ref[...], w1_ref[...],
              preferred_element_type=jnp.float32) + b1_ref[...]
  y_ref[...] = ds * h
  ds_ref[...] = ds


_RB = 2000


def _tc2_body(pa_ref, pb_ref, ds_ref, w0_ref, b0_ref, w1_ref, b1_ref,
              w2_ref, b2_ref, al_ref, g_ref, yc_ref, dp_ref, acc_ref):
  a = _softmax_head(al_ref[0, :], g_ref[0, 0])
  ds = ds_ref[...]
  z0 = _leaky(_norm(ds * pa_ref[0]))
  head0 = jnp.dot(z0, w0_ref[...], preferred_element_type=jnp.float32) + b0_ref[...]
  sb = pb_ref[0] + pb_ref[1]
  z1 = _leaky(_norm(0.5 * ds * sb))
  head1 = jnp.dot(z1, w1_ref[...], preferred_element_type=jnp.float32) + b1_ref[...]
  x_cat = jnp.concatenate([z0, z1], axis=1)
  dp = jnp.dot(x_cat, w2_ref[...], preferred_element_type=jnp.float32) + b2_ref[...]
  dp_ref[...] = dp
  yc_ref[...] = ds * dp
  acc_ref[...] = a[0] * _norm(head0) + a[1] * _norm(head1)


def _tc4_body(pc_ref, ds_ref, dp_ref, w_ref, b_ref, wd_ref, bd_ref,
              al_ref, g_ref, acc_in_ref, y_ref):
  a = _softmax_head(al_ref[0, :], g_ref[0, 0])
  sc = pc_ref[0, 0:_N, :] + pc_ref[1, 0:_N, :]
  ds = ds_ref[...]
  z = ds * sc + dp_ref[...]
  z2 = _leaky(_norm(z))
  head = jnp.dot(z2, w_ref[...], preferred_element_type=jnp.float32) + b_ref[...]
  out = acc_in_ref[...] + a[2] * _norm(head)
  out = out + a[3] * (ds * wd_ref[...] + bd_ref[...])
  m = jnp.max(out, axis=1, keepdims=True)
  sh = out - m
  y_ref[...] = sh - jnp.log(jnp.sum(jnp.exp(sh), axis=1, keepdims=True))


def _tc(body, out_shapes):
  return pl.pallas_call(body, out_shape=out_shapes)


def kernel(x, edge_index, W1, b1, W2, b2, W_out, b_out, Wd, bd, alpha, gamma):
  f32 = jnp.float32
  i32 = jnp.int32
  pad = _EP - _E
  src = jnp.concatenate([edge_index[0], jnp.arange(pad, dtype=i32) % _N])
  dst = jnp.concatenate(
      [edge_index[1], _N + (jnp.arange(pad, dtype=i32) % _NJUNK)])
  src = src.reshape(_ROWS, _C)
  dst = dst.reshape(_ROWS, _C)
  src_dual = jnp.stack([2 * src, 2 * src + 1])
  zeros_deg = jnp.zeros((_NPAD, _DEGF), f32)
  zeros64 = jnp.zeros((_NPAD, _HID), f32)
  ones_rows = jnp.ones((_C, _DEGF), f32)
  al_row = alpha.reshape(1, 4).astype(f32)
  g_sc = jnp.reshape(gamma, (1, 1)).astype(f32)

  degp = _deg_hist(ones_rows, src, dst, zeros_deg)

  y128, ds = _tc(_tc1_body, (
      jax.ShapeDtypeStruct((_N, _D_IN), f32),
      jax.ShapeDtypeStruct((_N, 1), f32),
  ))(x, W1, b1.reshape(1, _D_IN), degp)

  pa = _spmm_dual(y128.reshape(2 * _N, _HID), src_dual, dst, zeros64)

  pb = _spmm64(pa[1, 0:_N], src, dst, zeros64)

  nb = _N // _RB
  full = lambda *shape: pl.BlockSpec(shape, lambda i: (0,) * len(shape))
  yc, dp, acc12 = pl.pallas_call(
      _tc2_body,
      grid=(nb,),
      in_specs=[
          pl.BlockSpec((2, _RB, _HID), lambda i: (0, i, 0)),
          pl.BlockSpec((2, _RB, _HID), lambda i: (0, i, 0)),
          pl.BlockSpec((_RB, 1), lambda i: (i, 0)),
          full(_HID, _OUT), full(1, _OUT),
          full(_HID, _OUT), full(1, _OUT),
          full(2 * _HID, _HID), full(1, _HID),
          full(1, 4), full(1, 1),
      ],
      out_specs=[
          pl.BlockSpec((_RB, _HID), lambda i: (i, 0)),
          pl.BlockSpec((_RB, _HID), lambda i: (i, 0)),
          pl.BlockSpec((_RB, _OUT), lambda i: (i, 0)),
      ],
      out_shape=[
          jax.ShapeDtypeStruct((_N, _HID), f32),
          jax.ShapeDtypeStruct((_N, _HID), f32),
          jax.ShapeDtypeStruct((_N, _OUT), f32),
      ],
  )(pa, pb, ds, W_out[0], b_out[0].reshape(1, _OUT), W_out[1],
    b_out[1].reshape(1, _OUT), W2, b2.reshape(1, _HID), al_row, g_sc)

  pc = _spmm64(yc, src, dst, zeros64)

  y_hat = _tc(_tc4_body, jax.ShapeDtypeStruct((_N, _OUT), f32))(
      pc, ds, dp, W_out[0], b_out[0].reshape(1, _OUT), Wd,
      bd.reshape(1, _OUT), al_row, g_sc, acc12)

  return y_hat


# speedup vs baseline: 1.2054x; 1.0974x over previous
import functools

import jax
import jax.numpy as jnp
from jax import lax
from jax.experimental import pallas as pl
from jax.experimental.pallas import tpu as pltpu
from jax.experimental.pallas import tpu_sc as plsc

_N = 10000
_E = 320000
_D_IN = 128
_HID = 64
_OUT = 16

_NC = 2
_NS = 16
_NW = _NC * _NS
_C = 128
_K = 4
_EP = 327680
_NJUNK = 1024
_NPAD = _N + _NJUNK
_ROWS = _EP // _C

_DEGF = 16


def _make_spmm(F, mode):
  gather = mode != "deg"
  dual = mode == "dual"
  cpw = _ROWS // (_NS if dual else _NW)
  nbatch = cpw // _K
  npair = nbatch // 2
  rpt = _NPAD // _NS
  if dual:
    out_type = (jax.ShapeDtypeStruct((_NPAD, 2 * _HID), jnp.float32),
                jax.ShapeDtypeStruct((_NPAD, _HID), jnp.float32))
  else:
    out_type = jax.ShapeDtypeStruct((_NPAD, 2 * _HID), jnp.float32)
  mesh = plsc.VectorSubcoreMesh(core_axis_name="c", subcore_axis_name="s")
  if gather:
    idx_shape = (2, _K, _C)
  else:
    idx_shape = (2, 2 * _K, _C)
  scratch = [
      pltpu.VMEM(idx_shape, jnp.int32),
      pltpu.VMEM(idx_shape, jnp.int32),
      pltpu.VMEM((2, _K, _C, F), jnp.float32),
      pltpu.VMEM_SHARED((_NPAD, F), jnp.float32),
      pltpu.SemaphoreType.DMA,
      pltpu.SemaphoreType.DMA,
      pltpu.SemaphoreType.DMA,
      pltpu.SemaphoreType.DMA,
      pltpu.SemaphoreType.DMA,
  ]

  @functools.partial(
      pl.kernel,
      out_type=out_type,
      mesh=mesh,
      scratch_types=scratch,
      compiler_params=pltpu.CompilerParams(use_tc_tiling_on_sc=False),
  )
  def spmm(table_hbm, src_hbm, dst_hbm, zeros_hbm, *out_and_scratch):
    if dual:
      (out_hbm, out_t_hbm, src_b, dst_b, rows_b, acc_sh,
       g0, g1, s0, s1, isem) = out_and_scratch
    else:
      (out_hbm, src_b, dst_b, rows_b, acc_sh,
       g0, g1, s0, s1, isem) = out_and_scratch
    c = lax.axis_index("c")
    s = lax.axis_index("s")
    gsem = (g0, g1)
    ssem = (s0, s1)
    row0 = (s if dual else c * _NS + s) * cpw

    pltpu.sync_copy(zeros_hbm.at[pl.ds(s * rpt, rpt)],
                    acc_sh.at[pl.ds(s * rpt, rpt)])

    if not gather:
      pltpu.sync_copy(table_hbm, rows_b.at[0, 0])

    def idx_pair(pair, par):
      r = row0 + pair * 2 * _K
      dsts = pltpu.make_async_copy(dst_hbm.at[pl.ds(r, 2 * _K)],
                                   dst_b.at[par], isem)
      if not gather:
        return (dsts,)
      if dual:
        srcs = pltpu.make_async_copy(src_hbm.at[c, pl.ds(r, 2 * _K)],
                                     src_b.at[par], isem)
      else:
        srcs = pltpu.make_async_copy(src_hbm.at[pl.ds(r, 2 * _K)],
                                     src_b.at[par], isem)
      return (dsts, srcs)

    def fire_gathers(st, par=None):
      if gather:
        for b in range(_K):
          pltpu.async_copy(table_hbm.at[src_b.at[st, b]],
                           rows_b.at[st, b], gsem[st])

    def drain_gathers(st, par=None):
      if gather:
        for b in range(_K):
          pltpu.make_async_copy(table_hbm.at[src_b.at[st, b]],
                                rows_b.at[st, b], gsem[st]).wait()

    def fire_scatters(st, par=None):
      for b in range(_K):
        if gather:
          pltpu.async_copy(rows_b.at[st, b], acc_sh.at[dst_b.at[st, b]],
                           ssem[st], add=True)
        else:
          pltpu.async_copy(rows_b.at[0, 0],
                           acc_sh.at[dst_b.at[par, st * _K + b]],
                           ssem[st], add=True)

    def drain_scatters(st, par=None):
      for b in range(_K):
        if gather:
          pltpu.make_async_copy(rows_b.at[st, b], acc_sh.at[dst_b.at[st, b]],
                                ssem[st]).wait()
        else:
          pltpu.make_async_copy(rows_b.at[0, 0],
                                acc_sh.at[dst_b.at[par, st * _K + b]],
                                ssem[st]).wait()

    plsc.subcore_barrier()

    if gather:
      def load_idx(st, batch):
        r = row0 + batch * _K
        pltpu.sync_copy(dst_hbm.at[pl.ds(r, _K)], dst_b.at[st])
        pltpu.sync_copy(src_hbm.at[pl.ds(r, _K)], src_b.at[st])
        if dual:
          for b in range(_K):
            for v in range(_C // 16):
              sl = pl.ds(v * 16, 16)
              src_b[st, b, sl] = src_b[st, b, sl] * 2 + c

      load_idx(0, 0)
      fire_gathers(0)
      load_idx(1, 1)
      fire_gathers(1)

      def body(j, carry):
        drain_gathers(0)
        fire_scatters(0)
        drain_gathers(1)
        fire_scatters(1)

        @pl.when(j < npair - 1)
        def _next():
          drain_scatters(0)
          load_idx(0, 2 * j + 2)
          fire_gathers(0)
          drain_scatters(1)
          load_idx(1, 2 * j + 3)
          fire_gathers(1)

        return carry

      lax.fori_loop(0, npair, body, 0)
      drain_scatters(0)
      drain_scatters(1)
    else:
      descs0 = idx_pair(0, 0)
      for d in descs0:
        d.start()
      for d in descs0:
        d.wait()

      def body(j, carry):
        p = j % 2
        q = 1 - p
        fire_scatters(0, p)
        fire_scatters(1, p)

        @pl.when(j < npair - 1)
        def _next():
          descs = idx_pair(j + 1, q)
          for d in descs:
            d.start()
          drain_scatters(0, p)
          drain_scatters(1, p)
          for d in descs:
            d.wait()

        return carry

      lax.fori_loop(0, npair, body, 0)
      pf = (npair - 1) % 2
      drain_scatters(0, pf)
      drain_scatters(1, pf)

    plsc.subcore_barrier()
    rows_sl = pl.ds(s * rpt, rpt)
    pltpu.sync_copy(acc_sh.at[rows_sl],
                    out_hbm.at[rows_sl, pl.ds(c * _HID, F)])
    if dual:
      @pl.when(c == 1)
      def _dump_t():
        pltpu.sync_copy(acc_sh.at[rows_sl], out_t_hbm.at[rows_sl])

  return spmm


_spmm_dual = _make_spmm(_HID, "dual")
_spmm64 = _make_spmm(_HID, "split")
_deg_hist = _make_spmm(_DEGF, "deg")


def _norm(z):
  n = jnp.sqrt(jnp.sum(z * z, axis=1, keepdims=True))
  return z / jnp.maximum(n, 1e-12)


def _leaky(z):
  return jnp.where(z >= 0, z, 0.01 * z)


def _softmax_head(alpha_row, gamma_s):
  m = jnp.max(alpha_row)
  e = jnp.exp(alpha_row - m)
  return gamma_s * e / jnp.sum(e)


def _tc1_body(x_ref, w1_ref, b1_ref, degp_ref, y_ref, ds_ref):
  deg = degp_ref[0:_N, 0:1] + degp_ref[0:_N, _HID:_HID + 1]
  ds = jax.lax.rsqrt(jnp.maximum(deg, 1.0))
  h = jnp.dot(x_ref[...], w1_ref[...],
              preferred_element_type=jnp.float32) + b1_ref[...]
  y_ref[...] = ds * h
  ds_ref[...] = ds


_RB = 2000


def _tc2_body(pa_ref, pb_ref, ds_ref, w0_ref, b0_ref, w1_ref, b1_ref,
              w2_ref, b2_ref, al_ref, g_ref, yc_ref, dp_ref, acc_ref):
  a = _softmax_head(al_ref[0, :], g_ref[0, 0])
  ds = ds_ref[...]
  z0 = _leaky(_norm(ds * pa_ref[:, 0:_HID]))
  head0 = jnp.dot(z0, w0_ref[...], preferred_element_type=jnp.float32) + b0_ref[...]
  sb = pb_ref[:, 0:_HID] + pb_ref[:, _HID:]
  z1 = _leaky(_norm(0.5 * ds * sb))
  head1 = jnp.dot(z1, w1_ref[...], preferred_element_type=jnp.float32) + b1_ref[...]
  x_cat = jnp.concatenate([z0, z1], axis=1)
  dp = jnp.dot(x_cat, w2_ref[...], preferred_element_type=jnp.float32) + b2_ref[...]
  dp_ref[...] = dp
  yc_ref[...] = ds * dp
  acc_ref[...] = a[0] * _norm(head0) + a[1] * _norm(head1)


def _tc4_body(pc_ref, ds_ref, dp_ref, w_ref, b_ref, wd_ref, bd_ref,
              al_ref, g_ref, acc_in_ref, y_ref):
  a = _softmax_head(al_ref[0, :], g_ref[0, 0])
  sc = pc_ref[0:_N, 0:_HID] + pc_ref[0:_N, _HID:]
  ds = ds_ref[...]
  z = ds * sc + dp_ref[...]
  z2 = _leaky(_norm(z))
  head = jnp.dot(z2, w_ref[...], preferred_element_type=jnp.float32) + b_ref[...]
  out = acc_in_ref[...] + a[2] * _norm(head)
  out = out + a[3] * (ds * wd_ref[...] + bd_ref[...])
  m = jnp.max(out, axis=1, keepdims=True)
  sh = out - m
  y_ref[...] = sh - jnp.log(jnp.sum(jnp.exp(sh), axis=1, keepdims=True))


def _tc(body, out_shapes):
  return pl.pallas_call(body, out_shape=out_shapes)


def kernel(x, edge_index, W1, b1, W2, b2, W_out, b_out, Wd, bd, alpha, gamma):
  f32 = jnp.float32
  i32 = jnp.int32
  pad = _EP - _E
  pad_blk = jnp.stack([jnp.arange(pad, dtype=i32) % _N,
                       _N + (jnp.arange(pad, dtype=i32) % _NJUNK)])
  ep = jnp.concatenate([edge_index, pad_blk], axis=1).reshape(2, _ROWS, _C)
  src = ep[0]
  dst = ep[1]
  zeros_deg = jnp.zeros((_NPAD, _DEGF), f32)
  zeros64 = jnp.zeros((_NPAD, _HID), f32)
  ones_rows = jnp.ones((_C, _DEGF), f32)
  al_row = alpha.reshape(1, 4).astype(f32)
  g_sc = jnp.reshape(gamma, (1, 1)).astype(f32)

  degp = _deg_hist(ones_rows, src, dst, zeros_deg)

  y128, ds = _tc(_tc1_body, (
      jax.ShapeDtypeStruct((_N, _D_IN), f32),
      jax.ShapeDtypeStruct((_N, 1), f32),
  ))(x, W1, b1.reshape(1, _D_IN), degp)

  pa, t_tab = _spmm_dual(y128.reshape(2 * _N, _HID), src, dst, zeros64)

  pb = _spmm64(t_tab, src, dst, zeros64)

  nb = _N // _RB
  full = lambda *shape: pl.BlockSpec(shape, lambda i: (0,) * len(shape))
  yc, dp, acc12 = pl.pallas_call(
      _tc2_body,
      grid=(nb,),
      in_specs=[
          pl.BlockSpec((_RB, 2 * _HID), lambda i: (i, 0)),
          pl.BlockSpec((_RB, 2 * _HID), lambda i: (i, 0)),
          pl.BlockSpec((_RB, 1), lambda i: (i, 0)),
          full(_HID, _OUT), full(1, _OUT),
          full(_HID, _OUT), full(1, _OUT),
          full(2 * _HID, _HID), full(1, _HID),
          full(1, 4), full(1, 1),
      ],
      out_specs=[
          pl.BlockSpec((_RB, _HID), lambda i: (i, 0)),
          pl.BlockSpec((_RB, _HID), lambda i: (i, 0)),
          pl.BlockSpec((_RB, _OUT), lambda i: (i, 0)),
      ],
      out_shape=[
          jax.ShapeDtypeStruct((_N, _HID), f32),
          jax.ShapeDtypeStruct((_N, _HID), f32),
          jax.ShapeDtypeStruct((_N, _OUT), f32),
      ],
  )(pa, pb, ds, W_out[0], b_out[0].reshape(1, _OUT), W_out[1],
    b_out[1].reshape(1, _OUT), W2, b2.reshape(1, _HID), al_row, g_sc)

  pc = _spmm64(yc, src, dst, zeros64)

  y_hat = _tc(_tc4_body, jax.ShapeDtypeStruct((_N, _OUT), f32))(
      pc, ds, dp, W_out[0], b_out[0].reshape(1, _OUT), Wd,
      bd.reshape(1, _OUT), al_row, g_sc, acc12)

  return y_hat


# speedup vs baseline: 1.2161x; 1.0089x over previous
import functools

import numpy as np

import jax
import jax.numpy as jnp
from jax import lax
from jax.experimental import pallas as pl
from jax.experimental.pallas import tpu as pltpu
from jax.experimental.pallas import tpu_sc as plsc

_N = 10000
_E = 320000
_D_IN = 128
_HID = 64
_OUT = 16

_NC = 2
_NS = 16
_NW = _NC * _NS
_C = 128
_K = 4
_EP = 327680
_NJUNK = 1024
_NPAD = _N + _NJUNK
_ROWS = _EP // _C

_DEGF = 16


def _make_spmm(F, mode):
  gather = mode != "deg"
  dual = mode == "dual"
  cpw = _ROWS // (_NS if dual else _NW)
  nbatch = cpw // _K
  npair = nbatch // 2
  rpt = _NPAD // _NS
  if dual:
    out_type = (jax.ShapeDtypeStruct((_NPAD, 2 * _HID), jnp.float32),
                jax.ShapeDtypeStruct((_NPAD, _HID), jnp.float32))
  else:
    out_type = jax.ShapeDtypeStruct((_NPAD, 2 * _HID), jnp.float32)
  mesh = plsc.VectorSubcoreMesh(core_axis_name="c", subcore_axis_name="s")
  if gather:
    idx_shape = (2, _K, _C)
  else:
    idx_shape = (2, 2 * _K, _C)
  scratch = [
      pltpu.VMEM(idx_shape, jnp.int32),
      pltpu.VMEM(idx_shape, jnp.int32),
      pltpu.VMEM((2, _K, _C, F), jnp.float32),
      pltpu.VMEM_SHARED((_NPAD, F), jnp.float32),
      pltpu.SemaphoreType.DMA,
      pltpu.SemaphoreType.DMA,
      pltpu.SemaphoreType.DMA,
      pltpu.SemaphoreType.DMA,
      pltpu.SemaphoreType.DMA,
  ]

  @functools.partial(
      pl.kernel,
      out_type=out_type,
      mesh=mesh,
      scratch_types=scratch,
      compiler_params=pltpu.CompilerParams(use_tc_tiling_on_sc=False),
  )
  def spmm(table_hbm, src_hbm, dst_hbm, zeros_hbm, *out_and_scratch):
    if dual:
      (out_hbm, out_t_hbm, src_b, dst_b, rows_b, acc_sh,
       g0, g1, s0, s1, isem) = out_and_scratch
    else:
      (out_hbm, src_b, dst_b, rows_b, acc_sh,
       g0, g1, s0, s1, isem) = out_and_scratch
    c = lax.axis_index("c")
    s = lax.axis_index("s")
    gsem = (g0, g1)
    ssem = (s0, s1)
    row0 = (s if dual else c * _NS + s) * cpw

    pltpu.sync_copy(zeros_hbm.at[pl.ds(s * rpt, rpt)],
                    acc_sh.at[pl.ds(s * rpt, rpt)])

    if not gather:
      pltpu.sync_copy(table_hbm, rows_b.at[0, 0])

    def idx_pair(pair, par):
      r = row0 + pair * 2 * _K
      dsts = pltpu.make_async_copy(dst_hbm.at[pl.ds(r, 2 * _K)],
                                   dst_b.at[par], isem)
      if not gather:
        return (dsts,)
      if dual:
        srcs = pltpu.make_async_copy(src_hbm.at[c, pl.ds(r, 2 * _K)],
                                     src_b.at[par], isem)
      else:
        srcs = pltpu.make_async_copy(src_hbm.at[pl.ds(r, 2 * _K)],
                                     src_b.at[par], isem)
      return (dsts, srcs)

    def fire_gathers(st, par=None):
      if gather:
        for b in range(_K):
          pltpu.async_copy(table_hbm.at[src_b.at[st, b]],
                           rows_b.at[st, b], gsem[st])

    def drain_gathers(st, par=None):
      if gather:
        for b in range(_K):
          pltpu.make_async_copy(table_hbm.at[src_b.at[st, b]],
                                rows_b.at[st, b], gsem[st]).wait()

    def fire_scatters(st, par=None):
      for b in range(_K):
        if gather:
          pltpu.async_copy(rows_b.at[st, b], acc_sh.at[dst_b.at[st, b]],
                           ssem[st], add=True)
        else:
          pltpu.async_copy(rows_b.at[0, 0],
                           acc_sh.at[dst_b.at[par, st * _K + b]],
                           ssem[st], add=True)

    def drain_scatters(st, par=None):
      for b in range(_K):
        if gather:
          pltpu.make_async_copy(rows_b.at[st, b], acc_sh.at[dst_b.at[st, b]],
                                ssem[st]).wait()
        else:
          pltpu.make_async_copy(rows_b.at[0, 0],
                                acc_sh.at[dst_b.at[par, st * _K + b]],
                                ssem[st]).wait()

    plsc.subcore_barrier()

    if gather:
      def load_idx(st, batch):
        r = row0 + batch * _K
        pltpu.sync_copy(dst_hbm.at[pl.ds(r, _K)], dst_b.at[st])
        pltpu.sync_copy(src_hbm.at[pl.ds(r, _K)], src_b.at[st])
        if dual:
          for b in range(_K):
            for v in range(_C // 16):
              sl = pl.ds(v * 16, 16)
              src_b[st, b, sl] = src_b[st, b, sl] * 2 + c

      load_idx(0, 0)
      fire_gathers(0)
      load_idx(1, 1)
      fire_gathers(1)

      def body(j, carry):
        drain_gathers(0)
        fire_scatters(0)
        drain_gathers(1)
        fire_scatters(1)

        @pl.when(j < npair - 1)
        def _next():
          drain_scatters(0)
          load_idx(0, 2 * j + 2)
          fire_gathers(0)
          drain_scatters(1)
          load_idx(1, 2 * j + 3)
          fire_gathers(1)

        return carry

      lax.fori_loop(0, npair, body, 0)
      drain_scatters(0)
      drain_scatters(1)
    else:
      descs0 = idx_pair(0, 0)
      for d in descs0:
        d.start()
      for d in descs0:
        d.wait()

      def body(j, carry):
        p = j % 2
        q = 1 - p
        fire_scatters(0, p)
        fire_scatters(1, p)

        @pl.when(j < npair - 1)
        def _next():
          descs = idx_pair(j + 1, q)
          for d in descs:
            d.start()
          drain_scatters(0, p)
          drain_scatters(1, p)
          for d in descs:
            d.wait()

        return carry

      lax.fori_loop(0, npair, body, 0)
      pf = (npair - 1) % 2
      drain_scatters(0, pf)
      drain_scatters(1, pf)

    plsc.subcore_barrier()
    rows_sl = pl.ds(s * rpt, rpt)
    pltpu.sync_copy(acc_sh.at[rows_sl],
                    out_hbm.at[rows_sl, pl.ds(c * _HID, F)])
    if dual:
      @pl.when(c == 1)
      def _dump_t():
        pltpu.sync_copy(acc_sh.at[rows_sl], out_t_hbm.at[rows_sl])

  return spmm


_spmm_dual = _make_spmm(_HID, "dual")
_spmm64 = _make_spmm(_HID, "split")
_deg_hist = _make_spmm(_DEGF, "deg")


def _norm(z):
  n = jnp.sqrt(jnp.sum(z * z, axis=1, keepdims=True))
  return z / jnp.maximum(n, 1e-12)


def _leaky(z):
  return jnp.where(z >= 0, z, 0.01 * z)


def _softmax_head(alpha_row, gamma_s):
  m = jnp.max(alpha_row)
  e = jnp.exp(alpha_row - m)
  return gamma_s * e / jnp.sum(e)


def _tc1_body(x_ref, w1_ref, b1_ref, degp_ref, y_ref, ds_ref):
  deg = degp_ref[0:_N, 0:1] + degp_ref[0:_N, _HID:_HID + 1]
  ds = jax.lax.rsqrt(jnp.maximum(deg, 1.0))
  h = jnp.dot(x_ref[...], w1_ref[...],
              preferred_element_type=jnp.float32) + b1_ref[...]
  y_ref[...] = ds * h
  ds_ref[...] = ds


_RB = 2000


def _tc2_body(pa_ref, pb_ref, ds_ref, w0_ref, b0_ref, w1_ref, b1_ref,
              w2_ref, b2_ref, al_ref, g_ref, yc_ref, dp_ref, acc_ref):
  a = _softmax_head(al_ref[0, :], g_ref[0, 0])
  ds = ds_ref[...]
  z0 = _leaky(_norm(ds * pa_ref[:, 0:_HID]))
  head0 = jnp.dot(z0, w0_ref[...], preferred_element_type=jnp.float32) + b0_ref[...]
  sb = pb_ref[:, 0:_HID] + pb_ref[:, _HID:]
  z1 = _leaky(_norm(0.5 * ds * sb))
  head1 = jnp.dot(z1, w1_ref[...], preferred_element_type=jnp.float32) + b1_ref[...]
  x_cat = jnp.concatenate([z0, z1], axis=1)
  dp = jnp.dot(x_cat, w2_ref[...], preferred_element_type=jnp.float32) + b2_ref[...]
  dp_ref[...] = dp
  yc_ref[...] = ds * dp
  acc_ref[...] = a[0] * _norm(head0) + a[1] * _norm(head1)


def _tc4_body(pc_ref, ds_ref, dp_ref, w_ref, b_ref, wd_ref, bd_ref,
              al_ref, g_ref, acc_in_ref, y_ref):
  a = _softmax_head(al_ref[0, :], g_ref[0, 0])
  sc = pc_ref[0:_N, 0:_HID] + pc_ref[0:_N, _HID:]
  ds = ds_ref[...]
  z = ds * sc + dp_ref[...]
  z2 = _leaky(_norm(z))
  head = jnp.dot(z2, w_ref[...], preferred_element_type=jnp.float32) + b_ref[...]
  out = acc_in_ref[...] + a[2] * _norm(head)
  out = out + a[3] * (ds * wd_ref[...] + bd_ref[...])
  m = jnp.max(out, axis=1, keepdims=True)
  sh = out - m
  y_ref[...] = sh - jnp.log(jnp.sum(jnp.exp(sh), axis=1, keepdims=True))


def _tc(body, out_shapes):
  return pl.pallas_call(body, out_shape=out_shapes)


def kernel(x, edge_index, W1, b1, W2, b2, W_out, b_out, Wd, bd, alpha, gamma):
  f32 = jnp.float32
  i32 = jnp.int32
  pad = _EP - _E
  pad_blk = jnp.asarray(np.stack([np.arange(pad, dtype=np.int32) % _N,
                                  _N + (np.arange(pad) % _NJUNK)]).astype(
                                      np.int32))
  ep = jnp.concatenate([edge_index, pad_blk], axis=1).reshape(2, _ROWS, _C)
  src = ep[0]
  dst = ep[1]
  zeros_deg = jnp.zeros((_NPAD, _DEGF), f32)
  zeros64 = jnp.zeros((_NPAD, _HID), f32)
  ones_rows = jnp.ones((_C, _DEGF), f32)
  al_row = alpha.reshape(1, 4).astype(f32)
  g_sc = jnp.reshape(gamma, (1, 1)).astype(f32)

  degp = _deg_hist(ones_rows, src, dst, zeros_deg)

  y128, ds = _tc(_tc1_body, (
      jax.ShapeDtypeStruct((_N, _D_IN), f32),
      jax.ShapeDtypeStruct((_N, 1), f32),
  ))(x, W1, b1.reshape(1, _D_IN), degp)

  pa, t_tab = _spmm_dual(y128.reshape(2 * _N, _HID), src, dst, zeros64)

  pb = _spmm64(t_tab, src, dst, zeros64)

  nb = _N // _RB
  full = lambda *shape: pl.BlockSpec(shape, lambda i: (0,) * len(shape))
  yc, dp, acc12 = pl.pallas_call(
      _tc2_body,
      grid=(nb,),
      in_specs=[
          pl.BlockSpec((_RB, 2 * _HID), lambda i: (i, 0)),
          pl.BlockSpec((_RB, 2 * _HID), lambda i: (i, 0)),
          pl.BlockSpec((_RB, 1), lambda i: (i, 0)),
          full(_HID, _OUT), full(1, _OUT),
          full(_HID, _OUT), full(1, _OUT),
          full(2 * _HID, _HID), full(1, _HID),
          full(1, 4), full(1, 1),
      ],
      out_specs=[
          pl.BlockSpec((_RB, _HID), lambda i: (i, 0)),
          pl.BlockSpec((_RB, _HID), lambda i: (i, 0)),
          pl.BlockSpec((_RB, _OUT), lambda i: (i, 0)),
      ],
      out_shape=[
          jax.ShapeDtypeStruct((_N, _HID), f32),
          jax.ShapeDtypeStruct((_N, _HID), f32),
          jax.ShapeDtypeStruct((_N, _OUT), f32),
      ],
  )(pa, pb, ds, W_out[0], b_out[0].reshape(1, _OUT), W_out[1],
    b_out[1].reshape(1, _OUT), W2, b2.reshape(1, _HID), al_row, g_sc)

  pc = _spmm64(yc, src, dst, zeros64)

  y_hat = _tc(_tc4_body, jax.ShapeDtypeStruct((_N, _OUT), f32))(
      pc, ds, dp, W_out[0], b_out[0].reshape(1, _OUT), Wd,
      bd.reshape(1, _OUT), al_row, g_sc, acc12)

  return y_hat
